# baseline (speedup 1.0000x reference)
import functools
import jax
import jax.numpy as jnp
from jax import lax
from jax.experimental import pallas as pl
from jax.experimental.pallas import tpu as pltpu
from jax.experimental.pallas import tpu_sc as plsc

_NUM_BUCKETS = 12
_EMB_DIM = 32
_BOUNDS = (1.0, 2.0, 5.0, 10.0, 20.0, 50.0, 100.0, 200.0, 500.0, 1000.0, 2000.0)

_NC = 2
_NS = 16
_NW = _NC * _NS
_L = 16
_BT = 128
_TS = 5


@functools.lru_cache(maxsize=None)
def _build_sc_call(bsz, seq):
    btiles_per_w = bsz // (_NW * _BT)
    spans = seq // _TS
    kt = _EMB_DIM // 8

    @functools.partial(
        pl.kernel,
        mesh=plsc.VectorSubcoreMesh(core_axis_name="c", subcore_axis_name="s"),
        out_type=jax.ShapeDtypeStruct(
            (seq, kt, bsz // _BT, 8, _BT), jnp.float32
        ),
        scratch_types=[
            pltpu.VMEM((_NUM_BUCKETS, _EMB_DIM), jnp.float32),
            pltpu.VMEM((2 * _BT, seq), jnp.float32),
            pltpu.VMEM((2 * _TS, kt, 1, 8, _BT), jnp.float32),
            pltpu.SemaphoreType.DMA,
            pltpu.SemaphoreType.DMA,
            pltpu.SemaphoreType.DMA,
        ],
        compiler_params=pltpu.CompilerParams(needs_layout_passes=False),
    )
    def sc_call(
        amounts_hbm, emb_hbm, out_hbm, emb2, amt_v, blk_v, osem0, osem1, asem
    ):
        wid = lax.axis_index("s") * _NC + lax.axis_index("c")
        bt_base = wid * btiles_per_w
        pltpu.sync_copy(emb_hbm, emb2)
        lane = lax.iota(jnp.int32, _L)
        rclamp = jnp.minimum(lane, _NUM_BUCKETS - 1)
        cols = [
            plsc.load_gather(emb2, [rclamp, jnp.full((_L,), k, jnp.int32)])
            for k in range(_EMB_DIM)
        ]
        osems = (osem0, osem1)

        def amt_copy(i):
            return pltpu.make_async_copy(
                amounts_hbm.at[pl.ds((bt_base + i) * _BT, _BT), :],
                amt_v.at[pl.ds((i % 2) * _BT, _BT), :],
                asem,
            )

        def out_copy(i, h, half):
            return pltpu.make_async_copy(
                blk_v.at[pl.ds(half * _TS, _TS)],
                out_hbm.at[
                    pl.ds(h * _TS, _TS), :, pl.ds(bt_base + i, 1), :, :
                ],
                osems[half],
            )

        amt_copy(0).start()

        for i in range(btiles_per_w):
            arow = (i % 2) * _BT

            def span_pair(j, carry, i=i, arow=arow):
                for half in range(2):
                    h = 2 * j + half
                    if half == 0:
                        @pl.when(j == 0)
                        def _wait_amt():
                            amt_copy(i).wait()
                    elif i + 1 < btiles_per_w:
                        @pl.when(j == 0)
                        def _prefetch_amt():
                            amt_copy(i + 1).start()

                    @pl.when(j >= 1)
                    def _wait_same_btile(h=h, half=half):
                        out_copy(i, h - 2, half).wait()

                    if i > 0:
                        @pl.when(j == 0)
                        def _wait_prev_btile(half=half):
                            out_copy(i - 1, spans - 2 + half, half).wait()

                    base = half * _TS
                    t0 = h * _TS

                    @plsc.parallel_loop(0, _TS)
                    def trow(tl, base=base, t0=t0, arow=arow):
                        t = t0 + tl
                        row = base + tl
                        for bg in range(_BT // _L):
                            bvec = arow + bg * _L + lane
                            a = plsc.load_gather(
                                amt_v, [bvec, jnp.full((_L,), 0, jnp.int32) + t]
                            )
                            acc = jnp.zeros((_L,), jnp.int32)
                            for b in _BOUNDS:
                                acc = acc + jnp.where(a >= b, 1, 0)
                            for k in range(_EMB_DIM):
                                vals = cols[k].at[acc].get(
                                    mode="promise_in_bounds"
                                )
                                blk_v[
                                    row, k // 8, 0, k % 8, pl.ds(bg * _L, _L)
                                ] = vals

                    out_copy(i, h, half).start()
                return carry

            lax.fori_loop(0, spans // 2, span_pair, 0)

        out_copy(btiles_per_w - 1, spans - 2, 0).wait()
        out_copy(btiles_per_w - 1, spans - 1, 1).wait()

    return sc_call


def kernel(amounts, emb):
    bsz, seq = amounts.shape
    out5 = _build_sc_call(bsz, seq)(amounts, emb)
    out = out5.transpose(2, 4, 0, 1, 3).reshape(bsz, seq, _EMB_DIM)
    return out
